# baseline (speedup 1.0000x reference)
import functools

import jax
import jax.numpy as jnp
from jax import lax
from jax.experimental import pallas as pl
from jax.experimental.pallas import tpu as pltpu
from jax.experimental.pallas import tpu_sc as plsc

_D = 128
_NC = 2
_NS = 16
_NW = _NC * _NS
_CHUNK = 128
_NB = 4


@functools.cache
def _make_gather(B: int):
    rows_per_w = B // _NW
    chunks = rows_per_w // _CHUNK
    assert chunks % _NB == 0 and chunks > _NB
    mesh = plsc.VectorSubcoreMesh(core_axis_name="c", subcore_axis_name="s")

    @functools.partial(
        pl.kernel,
        out_type=jax.ShapeDtypeStruct((B, _D), jnp.float32),
        mesh=mesh,
        scratch_types=[
            pltpu.VMEM((rows_per_w,), jnp.int32),
            pltpu.VMEM((_NB, _CHUNK, _D), jnp.float32),
        ]
        + [pltpu.SemaphoreType.DMA] * (2 * _NB),
        compiler_params=pltpu.CompilerParams(use_tc_tiling_on_sc=True),
    )
    def gather_kernel(table_hbm, idx_hbm, out_hbm, idx_v, rows_v, *sems):
        gsem = sems[:_NB]
        wsem = sems[_NB:]
        wid = lax.axis_index("s") * _NC + lax.axis_index("c")
        base = wid * rows_per_w
        pltpu.sync_copy(idx_hbm.at[pl.ds(base, rows_per_w)], idx_v)

        def start_gather(c, b):
            pltpu.async_copy(
                table_hbm.at[idx_v.at[pl.ds(c * _CHUNK, _CHUNK)]],
                rows_v.at[b], gsem[b])

        def drain_gather(c, b):
            pltpu.make_async_copy(
                table_hbm.at[idx_v.at[pl.ds(c * _CHUNK, _CHUNK)]],
                rows_v.at[b], gsem[b]).wait()

        def start_write(c, b):
            return pltpu.async_copy(
                rows_v.at[b], out_hbm.at[pl.ds(base + c * _CHUNK, _CHUNK)],
                wsem[b])

        for b in range(_NB):
            start_gather(b, b)

        @pl.loop(0, chunks - _NB, step=_NB)
        def _(c):
            writes = []
            for b in range(_NB):
                drain_gather(c + b, b)
                writes.append(start_write(c + b, b))
            for b in range(_NB):
                writes[b].wait()
                start_gather(c + _NB + b, b)

        for b in range(_NB):
            c = chunks - _NB + b
            drain_gather(c, b)
            start_write(c, b).wait()

    return gather_kernel


def kernel(x, weight):
    B = x.shape[0] * x.shape[1]
    flat = x.reshape(B)
    table = jnp.pad(weight, ((0, 0), (0, _D - weight.shape[1])))
    out = _make_gather(B)(table, flat)
    return out[:, : weight.shape[1]].reshape(x.shape[0], x.shape[1], -1)


# speedup vs baseline: 1.1322x; 1.1322x over previous
import functools

import jax
import jax.numpy as jnp
from jax import lax
from jax.experimental import pallas as pl
from jax.experimental.pallas import tpu as pltpu
from jax.experimental.pallas import tpu_sc as plsc

_D = 64
_NC = 2
_NS = 16
_NW = _NC * _NS
_CHUNK = 256
_NB = 4


@functools.cache
def _make_gather(B: int):
    rows_per_w = B // _NW
    chunks = rows_per_w // _CHUNK
    assert chunks % _NB == 0 and chunks > _NB
    mesh = plsc.VectorSubcoreMesh(core_axis_name="c", subcore_axis_name="s")

    @functools.partial(
        pl.kernel,
        out_type=jax.ShapeDtypeStruct((B, _D), jnp.float32),
        mesh=mesh,
        scratch_types=[
            pltpu.VMEM((rows_per_w,), jnp.int32),
            pltpu.VMEM((_NB, _CHUNK, _D), jnp.float32),
        ]
        + [pltpu.SemaphoreType.DMA] * (2 * _NB),
        compiler_params=pltpu.CompilerParams(use_tc_tiling_on_sc=False),
    )
    def gather_kernel(table_hbm, idx_hbm, out_hbm, idx_v, rows_v, *sems):
        gsem = sems[:_NB]
        wsem = sems[_NB:]
        wid = lax.axis_index("s") * _NC + lax.axis_index("c")
        base = wid * rows_per_w
        pltpu.sync_copy(idx_hbm.at[pl.ds(base, rows_per_w)], idx_v)

        def start_gather(c, b):
            pltpu.async_copy(
                table_hbm.at[idx_v.at[pl.ds(c * _CHUNK, _CHUNK)]],
                rows_v.at[b], gsem[b])

        def drain_gather(c, b):
            pltpu.make_async_copy(
                table_hbm.at[idx_v.at[pl.ds(c * _CHUNK, _CHUNK)]],
                rows_v.at[b], gsem[b]).wait()

        def start_write(c, b):
            return pltpu.async_copy(
                rows_v.at[b], out_hbm.at[pl.ds(base + c * _CHUNK, _CHUNK)],
                wsem[b])

        for b in range(_NB):
            start_gather(b, b)

        @pl.loop(0, chunks - _NB, step=_NB)
        def _(c):
            writes = []
            for b in range(_NB):
                drain_gather(c + b, b)
                writes.append(start_write(c + b, b))
            for b in range(_NB):
                writes[b].wait()
                start_gather(c + _NB + b, b)

        for b in range(_NB):
            c = chunks - _NB + b
            drain_gather(c, b)
            start_write(c, b).wait()

    return gather_kernel


def kernel(x, weight):
    B = x.shape[0] * x.shape[1]
    flat = x.reshape(B)
    out = _make_gather(B)(weight, flat)
    return out.reshape(x.shape[0], x.shape[1], _D)
